# baseline (speedup 1.0000x reference)
import functools

import jax
import jax.numpy as jnp
from jax import lax
from jax.experimental import pallas as pl
from jax.experimental.pallas import tpu as pltpu
from jax.experimental.pallas import tpu_sc as plsc

N = 10000
E = 320000
D = 128
ED = 16
W = D // 2

NC = 2
NS = 16
CH = 128
NCHUNK = 160
NP2 = NCHUNK // 2
EPT = CH * NCHUNK
EPAD = EPT * NS
NPAD = 10112
RPT = NPAD // NS


def _edge_mlp_body(ea_ref, we_ref, be_ref, o_ref):
    r = (
        jnp.dot(ea_ref[...], we_ref[...], preferred_element_type=jnp.float32)
        + be_ref[...]
    )
    o_ref[0] = r[:, :W]
    o_ref[1] = r[:, W:]


def _edge_mlp(ea, we, be):
    m = 2048
    grid = EPAD // m
    return pl.pallas_call(
        _edge_mlp_body,
        grid=(grid,),
        in_specs=[
            pl.BlockSpec((m, ED), lambda i: (i, 0)),
            pl.BlockSpec((ED, D), lambda i: (0, 0)),
            pl.BlockSpec((1, D), lambda i: (0, 0)),
        ],
        out_specs=pl.BlockSpec((NC, m, W), lambda i: (0, i, 0)),
        out_shape=jax.ShapeDtypeStruct((NC, EPAD, W), jnp.float32),
    )(ea, we, be)


def _sc_body(h2_hbm, e_hbm, src_hbm, dst_hbm, out_hbm,
             isrc, idst, hbuf, ebuf, sbuf, acc,
             sem_i0, sem_i1, sem_s0, sem_s1):
    c = lax.axis_index("c")
    s = lax.axis_index("s")
    sem_in = (sem_i0, sem_i1)
    sem_sc = (sem_s0, sem_s1)

    pltpu.sync_copy(src_hbm.at[s], isrc)
    pltpu.sync_copy(dst_hbm.at[s], idst)

    @plsc.parallel_loop(0, NCHUNK)
    def _xform(r):
        for j in range(CH // 16):
            sl = pl.ds(j * 16, 16)
            isrc[r, sl] = isrc[r, sl] * 2 + c

    zero = jnp.zeros((16,), jnp.float32)

    @plsc.parallel_loop(0, CH)
    def _zrow(r):
        for j in range(W // 16):
            sbuf[0, r, pl.ds(j * 16, 16)] = zero

    for t in range(4):
        pltpu.sync_copy(sbuf.at[0], acc.at[pl.ds(s * RPT + t * CH, CH)])
    pltpu.sync_copy(sbuf.at[0, pl.ds(0, RPT - 4 * CH)],
                    acc.at[pl.ds(s * RPT + 4 * CH, RPT - 4 * CH)])
    plsc.subcore_barrier()

    base = s * EPT

    def _issue_in(g, b):
        pltpu.async_copy(h2_hbm.at[isrc.at[g]], hbuf.at[b], sem_in[b])
        pltpu.async_copy(e_hbm.at[c, pl.ds(base + g * CH, CH)], ebuf.at[b],
                         sem_in[b])

    def _wait_in(g, b):
        pltpu.make_async_copy(h2_hbm.at[isrc.at[g]], hbuf.at[b],
                              sem_in[b]).wait()
        pltpu.make_async_copy(e_hbm.at[c, pl.ds(base + g * CH, CH)],
                              ebuf.at[b], sem_in[b]).wait()

    def _compute(b):
        @plsc.parallel_loop(0, CH, unroll=2)
        def _row(r):
            for j in range(W // 16):
                sl = pl.ds(j * 16, 16)
                sbuf[b, r, sl] = jnp.maximum(hbuf[b, r, sl] + ebuf[b, r, sl],
                                             0.0)

    def _scatter(g, b):
        pltpu.sync_copy(sbuf.at[b], acc.at[idst.at[g]], add=True)

    _issue_in(0, 0)
    _issue_in(1, 1)

    def _pair(g2, carry):
        g = g2 * 2
        for b in range(2):
            gb = g + b

            _wait_in(gb, b)
            _compute(b)
            _scatter(gb, b)

            @pl.when(g2 < NP2 - 1)
            def _():
                _issue_in(gb + 2, b)

        return carry

    lax.fori_loop(0, NP2, _pair, 0)
    plsc.subcore_barrier()
    for t in range(4):
        r0 = s * RPT + t * CH
        pltpu.sync_copy(acc.at[pl.ds(r0, CH)], out_hbm.at[c, pl.ds(r0, CH)])
    r0 = s * RPT + 4 * CH
    pltpu.sync_copy(acc.at[pl.ds(r0, RPT - 4 * CH)],
                    out_hbm.at[c, pl.ds(r0, RPT - 4 * CH)])


_sc_message = functools.partial(
    pl.kernel,
    out_type=jax.ShapeDtypeStruct((NC, NPAD, W), jnp.float32),
    mesh=plsc.VectorSubcoreMesh(core_axis_name="c", subcore_axis_name="s"),
    compiler_params=pltpu.CompilerParams(use_tc_tiling_on_sc=False),
    scratch_types=[
        pltpu.VMEM((NCHUNK, CH), jnp.int32),
        pltpu.VMEM((NCHUNK, CH), jnp.int32),
        pltpu.VMEM((2, CH, W), jnp.float32),
        pltpu.VMEM((2, CH, W), jnp.float32),
        pltpu.VMEM((2, CH, W), jnp.float32),
        pltpu.VMEM_SHARED((NPAD, W), jnp.float32),
        pltpu.SemaphoreType.DMA,
        pltpu.SemaphoreType.DMA,
        pltpu.SemaphoreType.DMA,
        pltpu.SemaphoreType.DMA,
    ],
)(_sc_body)


def _node_mlp_body(h_ref, p_ref, w1_ref, b1_ref, w2_ref, b2_ref, o_ref):
    z = h_ref[...] + jnp.concatenate([p_ref[0], p_ref[1]], axis=1)
    t = jnp.maximum(
        jnp.dot(z, w1_ref[...], preferred_element_type=jnp.float32) + b1_ref[...],
        0.0,
    )
    o = jnp.dot(t, w2_ref[...], preferred_element_type=jnp.float32) + b2_ref[...]
    o_ref[...] = jnp.maximum(o, 0.0)


def _node_mlp(h, parts, w1, b1, w2, b2):
    m = 1024
    grid = (N + m - 1) // m
    return pl.pallas_call(
        _node_mlp_body,
        grid=(grid,),
        in_specs=[
            pl.BlockSpec((m, D), lambda i: (i, 0)),
            pl.BlockSpec((NC, m, W), lambda i: (0, i, 0)),
            pl.BlockSpec((D, D), lambda i: (0, 0)),
            pl.BlockSpec((1, D), lambda i: (0, 0)),
            pl.BlockSpec((D, D), lambda i: (0, 0)),
            pl.BlockSpec((1, D), lambda i: (0, 0)),
        ],
        out_specs=pl.BlockSpec((m, D), lambda i: (i, 0)),
        out_shape=jax.ShapeDtypeStruct((N, D), jnp.float32),
    )(h, parts, w1, b1, w2, b2)


def _final_body(h_ref, wt_ref, bt_ref, o_ref):
    t = jnp.sum(h_ref[...] * wt_ref[...], axis=1, keepdims=True) + bt_ref[...]
    o_ref[...] = 1.0 / (1.0 + jnp.exp(-t))


def _final(h, wt, bt):
    m = 1024
    grid = (N + m - 1) // m
    return pl.pallas_call(
        _final_body,
        grid=(grid,),
        in_specs=[
            pl.BlockSpec((m, D), lambda i: (i, 0)),
            pl.BlockSpec((1, D), lambda i: (0, 0)),
            pl.BlockSpec((1, 1), lambda i: (0, 0)),
        ],
        out_specs=pl.BlockSpec((m, 1), lambda i: (i, 0)),
        out_shape=jax.ShapeDtypeStruct((N, 1), jnp.float32),
    )(h, wt, bt)


def kernel(x, edge_index, edge_attr,
           We0, be0, W10, b10, W20, b20,
           We1, be1, W11, b11, W21, b21,
           We2, be2, W12, b12, W22, b22,
           Wout, bout):
    pad = EPAD - E
    src = jnp.concatenate([edge_index[0], jnp.zeros((pad,), jnp.int32)])
    dst = jnp.concatenate([edge_index[1], jnp.full((pad,), N, jnp.int32)])
    ea = jnp.concatenate([edge_attr, jnp.zeros((pad, ED), jnp.float32)])
    src = src.reshape(NS, NCHUNK, CH)
    dst = dst.reshape(NS, NCHUNK, CH)

    layers = [(We0, be0, W10, b10, W20, b20),
              (We1, be1, W11, b11, W21, b21),
              (We2, be2, W12, b12, W22, b22)]
    h = x
    for (we, be, w1, b1, w2, b2) in layers:
        e = _edge_mlp(ea, we, be.reshape(1, D))
        parts = _sc_message(h.reshape(2 * N, W), e, src, dst)
        h = _node_mlp(h, parts, w1, b1.reshape(1, D), w2, b2.reshape(1, D))
    out = _final(h, Wout.reshape(1, D), bout.reshape(1, 1))
    return jnp.squeeze(out, axis=-1)


# speedup vs baseline: 1.0039x; 1.0039x over previous
import functools

import jax
import jax.numpy as jnp
from jax import lax
from jax.experimental import pallas as pl
from jax.experimental.pallas import tpu as pltpu
from jax.experimental.pallas import tpu_sc as plsc

N = 10000
E = 320000
D = 128
ED = 16
W = D // 2

NC = 2
NS = 16
CH = 128
NCHUNK = 160
NP2 = NCHUNK // 2
EPT = CH * NCHUNK
EPAD = EPT * NS
NPAD = 10112
RPT = NPAD // NS


def _edge_mlp_body(ea_ref, we_ref, be_ref, o_ref):
    r = (
        jnp.dot(ea_ref[...], we_ref[...], preferred_element_type=jnp.float32)
        + be_ref[...]
    )
    o_ref[0] = r[:, :W]
    o_ref[1] = r[:, W:]


def _edge_mlp(ea, we, be):
    m = 2048
    grid = EPAD // m
    return pl.pallas_call(
        _edge_mlp_body,
        grid=(grid,),
        in_specs=[
            pl.BlockSpec((m, ED), lambda i: (i, 0)),
            pl.BlockSpec((ED, D), lambda i: (0, 0)),
            pl.BlockSpec((1, D), lambda i: (0, 0)),
        ],
        out_specs=pl.BlockSpec((NC, m, W), lambda i: (0, i, 0)),
        out_shape=jax.ShapeDtypeStruct((NC, EPAD, W), jnp.float32),
    )(ea, we, be)


def _sc_body(h2_hbm, e_hbm, src_hbm, dst_hbm, out_hbm,
             isrc, idst, hbuf, ebuf, sbuf, acc,
             sem_i0, sem_i1, sem_s0, sem_s1):
    c = lax.axis_index("c")
    s = lax.axis_index("s")
    sem_in = (sem_i0, sem_i1)
    sem_sc = (sem_s0, sem_s1)

    pltpu.sync_copy(src_hbm.at[s], isrc)
    pltpu.sync_copy(dst_hbm.at[s], idst)

    @plsc.parallel_loop(0, NCHUNK)
    def _xform(r):
        for j in range(CH // 16):
            sl = pl.ds(j * 16, 16)
            isrc[r, sl] = isrc[r, sl] * 2 + c

    zero = jnp.zeros((16,), jnp.float32)

    @plsc.parallel_loop(0, CH)
    def _zrow(r):
        for j in range(W // 16):
            sbuf[0, r, pl.ds(j * 16, 16)] = zero

    for t in range(4):
        pltpu.sync_copy(sbuf.at[0], acc.at[pl.ds(s * RPT + t * CH, CH)])
    pltpu.sync_copy(sbuf.at[0, pl.ds(0, RPT - 4 * CH)],
                    acc.at[pl.ds(s * RPT + 4 * CH, RPT - 4 * CH)])
    plsc.subcore_barrier()

    base = s * EPT

    def _issue_in(g, b):
        pltpu.async_copy(h2_hbm.at[isrc.at[g]], hbuf.at[b], sem_in[b])
        pltpu.async_copy(e_hbm.at[c, pl.ds(base + g * CH, CH)], ebuf.at[b],
                         sem_in[b])

    def _wait_in(g, b):
        pltpu.make_async_copy(h2_hbm.at[isrc.at[g]], hbuf.at[b],
                              sem_in[b]).wait()
        pltpu.make_async_copy(e_hbm.at[c, pl.ds(base + g * CH, CH)],
                              ebuf.at[b], sem_in[b]).wait()

    def _compute(b):
        @plsc.parallel_loop(0, CH, unroll=2)
        def _row(r):
            for j in range(W // 16):
                sl = pl.ds(j * 16, 16)
                sbuf[b, r, sl] = jnp.maximum(hbuf[b, r, sl] + ebuf[b, r, sl],
                                             0.0)

    _issue_in(0, 0)
    _issue_in(1, 1)

    def _pair(g2, carry):
        g = g2 * 2

        _wait_in(g, 0)
        _compute(0)
        cp0 = pltpu.async_copy(sbuf.at[0], acc.at[idst.at[g]], sem_sc[0],
                               add=True)

        @pl.when(g2 < NP2 - 1)
        def _():
            _issue_in(g + 2, 0)

        _wait_in(g + 1, 1)
        _compute(1)
        cp1 = pltpu.async_copy(sbuf.at[1], acc.at[idst.at[g + 1]], sem_sc[1],
                               add=True)

        @pl.when(g2 < NP2 - 1)
        def _():
            _issue_in(g + 3, 1)

        cp0.wait()
        cp1.wait()
        return carry

    lax.fori_loop(0, NP2, _pair, 0)
    plsc.subcore_barrier()
    for t in range(4):
        r0 = s * RPT + t * CH
        pltpu.sync_copy(acc.at[pl.ds(r0, CH)], out_hbm.at[c, pl.ds(r0, CH)])
    r0 = s * RPT + 4 * CH
    pltpu.sync_copy(acc.at[pl.ds(r0, RPT - 4 * CH)],
                    out_hbm.at[c, pl.ds(r0, RPT - 4 * CH)])


_sc_message = functools.partial(
    pl.kernel,
    out_type=jax.ShapeDtypeStruct((NC, NPAD, W), jnp.float32),
    mesh=plsc.VectorSubcoreMesh(core_axis_name="c", subcore_axis_name="s"),
    compiler_params=pltpu.CompilerParams(use_tc_tiling_on_sc=False),
    scratch_types=[
        pltpu.VMEM((NCHUNK, CH), jnp.int32),
        pltpu.VMEM((NCHUNK, CH), jnp.int32),
        pltpu.VMEM((2, CH, W), jnp.float32),
        pltpu.VMEM((2, CH, W), jnp.float32),
        pltpu.VMEM((2, CH, W), jnp.float32),
        pltpu.VMEM_SHARED((NPAD, W), jnp.float32),
        pltpu.SemaphoreType.DMA,
        pltpu.SemaphoreType.DMA,
        pltpu.SemaphoreType.DMA,
        pltpu.SemaphoreType.DMA,
    ],
)(_sc_body)


def _node_mlp_body(h_ref, p_ref, w1_ref, b1_ref, w2_ref, b2_ref, o_ref):
    z = h_ref[...] + jnp.concatenate([p_ref[0], p_ref[1]], axis=1)
    t = jnp.maximum(
        jnp.dot(z, w1_ref[...], preferred_element_type=jnp.float32) + b1_ref[...],
        0.0,
    )
    o = jnp.dot(t, w2_ref[...], preferred_element_type=jnp.float32) + b2_ref[...]
    o_ref[...] = jnp.maximum(o, 0.0)


def _node_mlp(h, parts, w1, b1, w2, b2):
    m = 1024
    grid = (N + m - 1) // m
    return pl.pallas_call(
        _node_mlp_body,
        grid=(grid,),
        in_specs=[
            pl.BlockSpec((m, D), lambda i: (i, 0)),
            pl.BlockSpec((NC, m, W), lambda i: (0, i, 0)),
            pl.BlockSpec((D, D), lambda i: (0, 0)),
            pl.BlockSpec((1, D), lambda i: (0, 0)),
            pl.BlockSpec((D, D), lambda i: (0, 0)),
            pl.BlockSpec((1, D), lambda i: (0, 0)),
        ],
        out_specs=pl.BlockSpec((m, D), lambda i: (i, 0)),
        out_shape=jax.ShapeDtypeStruct((N, D), jnp.float32),
    )(h, parts, w1, b1, w2, b2)


def _final_body(h_ref, wt_ref, bt_ref, o_ref):
    t = jnp.sum(h_ref[...] * wt_ref[...], axis=1, keepdims=True) + bt_ref[...]
    o_ref[...] = 1.0 / (1.0 + jnp.exp(-t))


def _final(h, wt, bt):
    m = 1024
    grid = (N + m - 1) // m
    return pl.pallas_call(
        _final_body,
        grid=(grid,),
        in_specs=[
            pl.BlockSpec((m, D), lambda i: (i, 0)),
            pl.BlockSpec((1, D), lambda i: (0, 0)),
            pl.BlockSpec((1, 1), lambda i: (0, 0)),
        ],
        out_specs=pl.BlockSpec((m, 1), lambda i: (i, 0)),
        out_shape=jax.ShapeDtypeStruct((N, 1), jnp.float32),
    )(h, wt, bt)


def kernel(x, edge_index, edge_attr,
           We0, be0, W10, b10, W20, b20,
           We1, be1, W11, b11, W21, b21,
           We2, be2, W12, b12, W22, b22,
           Wout, bout):
    pad = EPAD - E
    src = jnp.concatenate([edge_index[0], jnp.zeros((pad,), jnp.int32)])
    dst = jnp.concatenate([edge_index[1], jnp.full((pad,), N, jnp.int32)])
    ea = jnp.concatenate([edge_attr, jnp.zeros((pad, ED), jnp.float32)])
    src = src.reshape(NS, NCHUNK, CH)
    dst = dst.reshape(NS, NCHUNK, CH)

    layers = [(We0, be0, W10, b10, W20, b20),
              (We1, be1, W11, b11, W21, b21),
              (We2, be2, W12, b12, W22, b22)]
    es = [_edge_mlp(ea, we, be.reshape(1, D)) for (we, be, *_rest) in layers]
    h = x
    for e, (we, be, w1, b1, w2, b2) in zip(es, layers):
        parts = _sc_message(h.reshape(2 * N, W), e, src, dst)
        h = _node_mlp(h, parts, w1, b1.reshape(1, D), w2, b2.reshape(1, D))
    out = _final(h, Wout.reshape(1, D), bout.reshape(1, 1))
    return jnp.squeeze(out, axis=-1)


# speedup vs baseline: 1.2826x; 1.2775x over previous
import functools

import jax
import jax.numpy as jnp
from jax import lax
from jax.experimental import pallas as pl
from jax.experimental.pallas import tpu as pltpu
from jax.experimental.pallas import tpu_sc as plsc

N = 10000
E = 320000
D = 128
ED = 16

NC = 2
NS = 16
NW = NC * NS
CH = 64
NCHUNK = 160
NP2 = NCHUNK // 2
EPW = CH * NCHUNK
EPAD = EPW * NW
NPAD = 10112
RPT = NPAD // NS


def _edge_mlp_body(ea_ref, we_ref, be_ref, o_ref):
    o_ref[...] = (
        jnp.dot(ea_ref[...], we_ref[...], preferred_element_type=jnp.float32)
        + be_ref[...]
    )


def _edge_mlp(ea, we, be):
    m = 2048
    grid = EPAD // m
    return pl.pallas_call(
        _edge_mlp_body,
        grid=(grid,),
        in_specs=[
            pl.BlockSpec((m, ED), lambda i: (i, 0)),
            pl.BlockSpec((ED, D), lambda i: (0, 0)),
            pl.BlockSpec((1, D), lambda i: (0, 0)),
        ],
        out_specs=pl.BlockSpec((m, D), lambda i: (i, 0)),
        out_shape=jax.ShapeDtypeStruct((EPAD, D), jnp.float32),
    )(ea, we, be)


SUP = 8
NSUP = NCHUNK // SUP


def _sc_body(h_hbm, e_hbm, src_hbm, dst_hbm, out_hbm,
             sring, dring, hbuf, ebuf, sbuf, acc,
             sem_sx0, sem_sx1, sem_sx2, sem_sx3,
             sem_dx0, sem_dx1, sem_dx2, sem_dx3,
             sem_i0, sem_i1, sem_s0, sem_s1):
    c = lax.axis_index("c")
    s = lax.axis_index("s")
    wid = c * NS + s
    sem_sx = (sem_sx0, sem_sx1, sem_sx2, sem_sx3)
    sem_dx = (sem_dx0, sem_dx1, sem_dx2, sem_dx3)
    sem_in = (sem_i0, sem_i1)
    sem_sc = (sem_s0, sem_s1)

    zero = jnp.zeros((16,), jnp.float32)

    @plsc.parallel_loop(0, CH)
    def _zrow(r):
        for j in range(D // 16):
            sbuf[0, r, pl.ds(j * 16, 16)] = zero

    for t in range(RPT // CH):
        pltpu.sync_copy(sbuf.at[0], acc.at[pl.ds(s * RPT + t * CH, CH)])
    tail = RPT - (RPT // CH) * CH
    if tail:
        pltpu.sync_copy(sbuf.at[0, pl.ds(0, tail)],
                        acc.at[pl.ds(s * RPT + (RPT // CH) * CH, tail)])
    plsc.subcore_barrier()

    def _issue_srcx(g, q):
        pltpu.async_copy(src_hbm.at[wid, g], sring.at[q], sem_sx[q])

    def _wait_srcx(g, q):
        pltpu.make_async_copy(src_hbm.at[wid, g], sring.at[q],
                              sem_sx[q]).wait()

    def _issue_dstx(g, q):
        pltpu.async_copy(dst_hbm.at[wid, g], dring.at[q], sem_dx[q])

    def _wait_dstx(g, q):
        pltpu.make_async_copy(dst_hbm.at[wid, g], dring.at[q],
                              sem_dx[q]).wait()

    def _issue_in(g, q, b):
        pltpu.async_copy(h_hbm.at[sring.at[q]], hbuf.at[b], sem_in[b])
        pltpu.async_copy(e_hbm.at[pl.ds(wid * EPW + g * CH, CH)], ebuf.at[b],
                         sem_in[b])

    def _wait_in(g, q, b):
        pltpu.make_async_copy(h_hbm.at[sring.at[q]], hbuf.at[b],
                              sem_in[b]).wait()
        pltpu.make_async_copy(e_hbm.at[pl.ds(wid * EPW + g * CH, CH)],
                              ebuf.at[b], sem_in[b]).wait()

    def _compute(b):
        @plsc.parallel_loop(0, CH, unroll=2)
        def _row(r):
            for j in range(D // 16):
                sl = pl.ds(j * 16, 16)
                sbuf[b, r, sl] = jnp.maximum(hbuf[b, r, sl] + ebuf[b, r, sl],
                                             0.0)

    for g in range(4):
        _issue_srcx(g, g)
    for g in range(2):
        _issue_dstx(g, g)
    for g in range(2):
        _wait_srcx(g, g)
        _issue_in(g, g, g)

    def _super(i, carry):
        g0 = i * SUP
        cps = [None] * SUP
        for k in range(SUP):
            gk = g0 + k
            b = k % 2
            q = k % 4

            if k >= 2:
                cps[k - 2].wait()

            _wait_in(gk, q, b)
            _compute(b)
            _wait_dstx(gk, q)
            cps[k] = pltpu.async_copy(sbuf.at[b], acc.at[dring.at[q]],
                                      sem_sc[b], add=True)

            def _dst_refill(gg=gk + 2, qq=(k + 2) % 4):
                _issue_dstx(gg, qq)

            if k < SUP - 2:
                _dst_refill()
            else:
                @pl.when(i < NSUP - 1)
                def _():
                    _dst_refill()

            def _in_prefetch(gg=gk + 2, qq=(k + 2) % 4, bb=b):
                _wait_srcx(gg, qq)
                _issue_in(gg, qq, bb)

            if k < SUP - 2:
                _in_prefetch()
            else:
                @pl.when(i < NSUP - 1)
                def _():
                    _in_prefetch()

            def _src_refill(gg=gk + 4, qq=k % 4):
                _issue_srcx(gg, qq)

            if k < SUP - 4:
                _src_refill()
            else:
                @pl.when(i < NSUP - 1)
                def _():
                    _src_refill()

        cps[SUP - 2].wait()
        cps[SUP - 1].wait()
        return carry

    lax.fori_loop(0, NSUP, _super, 0)
    plsc.subcore_barrier()
    for t in range(RPT // CH):
        r0 = s * RPT + t * CH
        pltpu.sync_copy(acc.at[pl.ds(r0, CH)], out_hbm.at[c, pl.ds(r0, CH)])
    if tail:
        r0 = s * RPT + (RPT // CH) * CH
        pltpu.sync_copy(acc.at[pl.ds(r0, tail)],
                        out_hbm.at[c, pl.ds(r0, tail)])


_sc_message = functools.partial(
    pl.kernel,
    out_type=jax.ShapeDtypeStruct((NC, NPAD, D), jnp.float32),
    mesh=plsc.VectorSubcoreMesh(core_axis_name="c", subcore_axis_name="s"),
    scratch_types=[
        pltpu.VMEM((4, CH), jnp.int32),
        pltpu.VMEM((4, CH), jnp.int32),
        pltpu.VMEM((2, CH, D), jnp.float32),
        pltpu.VMEM((2, CH, D), jnp.float32),
        pltpu.VMEM((2, CH, D), jnp.float32),
        pltpu.VMEM_SHARED((NPAD, D), jnp.float32),
    ] + [pltpu.SemaphoreType.DMA] * 12,
)(_sc_body)


def _node_mlp_body(h_ref, p_ref, w1_ref, b1_ref, w2_ref, b2_ref, o_ref):
    z = h_ref[...] + p_ref[0] + p_ref[1]
    t = jnp.maximum(
        jnp.dot(z, w1_ref[...], preferred_element_type=jnp.float32) + b1_ref[...],
        0.0,
    )
    o = jnp.dot(t, w2_ref[...], preferred_element_type=jnp.float32) + b2_ref[...]
    o_ref[...] = jnp.maximum(o, 0.0)


def _node_mlp(h, parts, w1, b1, w2, b2):
    m = 1024
    grid = (N + m - 1) // m
    return pl.pallas_call(
        _node_mlp_body,
        grid=(grid,),
        in_specs=[
            pl.BlockSpec((m, D), lambda i: (i, 0)),
            pl.BlockSpec((NC, m, D), lambda i: (0, i, 0)),
            pl.BlockSpec((D, D), lambda i: (0, 0)),
            pl.BlockSpec((1, D), lambda i: (0, 0)),
            pl.BlockSpec((D, D), lambda i: (0, 0)),
            pl.BlockSpec((1, D), lambda i: (0, 0)),
        ],
        out_specs=pl.BlockSpec((m, D), lambda i: (i, 0)),
        out_shape=jax.ShapeDtypeStruct((N, D), jnp.float32),
    )(h, parts, w1, b1, w2, b2)


def _final_body(h_ref, wt_ref, bt_ref, o_ref):
    t = jnp.sum(h_ref[...] * wt_ref[...], axis=1, keepdims=True) + bt_ref[...]
    o_ref[...] = 1.0 / (1.0 + jnp.exp(-t))


def _final(h, wt, bt):
    m = 1024
    grid = (N + m - 1) // m
    return pl.pallas_call(
        _final_body,
        grid=(grid,),
        in_specs=[
            pl.BlockSpec((m, D), lambda i: (i, 0)),
            pl.BlockSpec((1, D), lambda i: (0, 0)),
            pl.BlockSpec((1, 1), lambda i: (0, 0)),
        ],
        out_specs=pl.BlockSpec((m, 1), lambda i: (i, 0)),
        out_shape=jax.ShapeDtypeStruct((N, 1), jnp.float32),
    )(h, wt, bt)


def kernel(x, edge_index, edge_attr,
           We0, be0, W10, b10, W20, b20,
           We1, be1, W11, b11, W21, b21,
           We2, be2, W12, b12, W22, b22,
           Wout, bout):
    pad = EPAD - E
    src = jnp.concatenate([edge_index[0], jnp.zeros((pad,), jnp.int32)])
    dst = jnp.concatenate([edge_index[1], jnp.full((pad,), N, jnp.int32)])
    ea = jnp.concatenate([edge_attr, jnp.zeros((pad, ED), jnp.float32)])
    src = src.reshape(NW, NCHUNK, CH)
    dst = dst.reshape(NW, NCHUNK, CH)

    layers = [(We0, be0, W10, b10, W20, b20),
              (We1, be1, W11, b11, W21, b21),
              (We2, be2, W12, b12, W22, b22)]
    es = [_edge_mlp(ea, we, be.reshape(1, D)) for (we, be, *_rest) in layers]
    h = x
    for e, (we, be, w1, b1, w2, b2) in zip(es, layers):
        parts = _sc_message(h, e, src, dst)
        h = _node_mlp(h, parts, w1, b1.reshape(1, D), w2, b2.reshape(1, D))
    out = _final(h, Wout.reshape(1, D), bout.reshape(1, 1))
    return jnp.squeeze(out, axis=-1)


# speedup vs baseline: 1.4376x; 1.1208x over previous
import functools

import jax
import jax.numpy as jnp
from jax import lax
from jax.experimental import pallas as pl
from jax.experimental.pallas import tpu as pltpu
from jax.experimental.pallas import tpu_sc as plsc

N = 10000
E = 320000
D = 128
ED = 16

NC = 2
NS = 16
NW = NC * NS
CH = 128
NCHUNK = 79
EPW = CH * NCHUNK
EPAD = EPW * NW
NPAD = 10240
RPT = NPAD // NS


def _edge_mlp_body(ea_ref, we_ref, be_ref, o_ref):
    o_ref[...] = (
        jnp.dot(ea_ref[...], we_ref[...], preferred_element_type=jnp.float32)
        + be_ref[...]
    )


def _edge_mlp(ea, we, be):
    m = 2048
    grid = EPAD // m
    return pl.pallas_call(
        _edge_mlp_body,
        grid=(grid,),
        in_specs=[
            pl.BlockSpec((m, ED), lambda i: (i, 0)),
            pl.BlockSpec((ED, D), lambda i: (0, 0)),
            pl.BlockSpec((1, D), lambda i: (0, 0)),
        ],
        out_specs=pl.BlockSpec((m, D), lambda i: (i, 0)),
        out_shape=jax.ShapeDtypeStruct((EPAD, D), jnp.float32),
    )(ea, we, be)


def _sc_body(h_hbm, e_hbm, sd_hbm, out_hbm,
             isd, ebuf, hbuf, acc, sem_g, sem_e):
    c = lax.axis_index("c")
    s = lax.axis_index("s")
    wid = c * NS + s
    base = wid * EPW

    zero = jnp.zeros((16,), jnp.float32)

    def _zrow(r, carry):
        for j in range(D // 16):
            hbuf[r, pl.ds(j * 16, 16)] = zero
        return carry

    lax.fori_loop(0, CH, _zrow, 0)
    for t in range(RPT // CH):
        pltpu.sync_copy(hbuf, acc.at[pl.ds(s * RPT + t * CH, CH)])
    plsc.subcore_barrier()

    def _chunk(g, carry):
        off = base + g * CH
        pltpu.sync_copy(sd_hbm.at[wid, g], isd)
        cp_g = pltpu.async_copy(h_hbm.at[isd.at[0]], hbuf, sem_g)
        cp_e = pltpu.async_copy(e_hbm.at[pl.ds(off, CH)], ebuf, sem_e)
        cp_g.wait()
        cp_e.wait()

        def _row(r, rc):
            for j in range(D // 16):
                sl = pl.ds(j * 16, 16)
                ebuf[r, sl] = jnp.maximum(hbuf[r, sl] + ebuf[r, sl], 0.0)
            return rc

        lax.fori_loop(0, CH, _row, 0)
        pltpu.sync_copy(ebuf, acc.at[isd.at[1]], add=True)
        return carry

    lax.fori_loop(0, NCHUNK, _chunk, 0)
    plsc.subcore_barrier()
    for t in range(RPT // CH):
        r0 = s * RPT + t * CH
        pltpu.sync_copy(acc.at[pl.ds(r0, CH)], out_hbm.at[c, pl.ds(r0, CH)])


_sc_message = functools.partial(
    pl.kernel,
    out_type=jax.ShapeDtypeStruct((NC, NPAD, D), jnp.float32),
    mesh=plsc.VectorSubcoreMesh(core_axis_name="c", subcore_axis_name="s"),
    scratch_types=[
        pltpu.VMEM((2, CH), jnp.int32),
        pltpu.VMEM((CH, D), jnp.float32),
        pltpu.VMEM((CH, D), jnp.float32),
        pltpu.VMEM_SHARED((NPAD, D), jnp.float32),
        pltpu.SemaphoreType.DMA,
        pltpu.SemaphoreType.DMA,
    ],
)(_sc_body)


def _node_mlp_body(h_ref, p_ref, w1_ref, b1_ref, w2_ref, b2_ref, o_ref):
    z = h_ref[...] + p_ref[0] + p_ref[1]
    t = jnp.maximum(
        jnp.dot(z, w1_ref[...], preferred_element_type=jnp.float32) + b1_ref[...],
        0.0,
    )
    o = jnp.dot(t, w2_ref[...], preferred_element_type=jnp.float32) + b2_ref[...]
    o_ref[...] = jnp.maximum(o, 0.0)


def _node_mlp(h, parts, w1, b1, w2, b2):
    m = 1024
    grid = (N + m - 1) // m
    return pl.pallas_call(
        _node_mlp_body,
        grid=(grid,),
        in_specs=[
            pl.BlockSpec((m, D), lambda i: (i, 0)),
            pl.BlockSpec((NC, m, D), lambda i: (0, i, 0)),
            pl.BlockSpec((D, D), lambda i: (0, 0)),
            pl.BlockSpec((1, D), lambda i: (0, 0)),
            pl.BlockSpec((D, D), lambda i: (0, 0)),
            pl.BlockSpec((1, D), lambda i: (0, 0)),
        ],
        out_specs=pl.BlockSpec((m, D), lambda i: (i, 0)),
        out_shape=jax.ShapeDtypeStruct((N, D), jnp.float32),
    )(h, parts, w1, b1, w2, b2)


def _final_body(h_ref, wt_ref, bt_ref, o_ref):
    t = jnp.sum(h_ref[...] * wt_ref[...], axis=1, keepdims=True) + bt_ref[...]
    o_ref[...] = 1.0 / (1.0 + jnp.exp(-t))


def _final(h, wt, bt):
    m = 1024
    grid = (N + m - 1) // m
    return pl.pallas_call(
        _final_body,
        grid=(grid,),
        in_specs=[
            pl.BlockSpec((m, D), lambda i: (i, 0)),
            pl.BlockSpec((1, D), lambda i: (0, 0)),
            pl.BlockSpec((1, 1), lambda i: (0, 0)),
        ],
        out_specs=pl.BlockSpec((m, 1), lambda i: (i, 0)),
        out_shape=jax.ShapeDtypeStruct((N, 1), jnp.float32),
    )(h, wt, bt)


def kernel(x, edge_index, edge_attr,
           We0, be0, W10, b10, W20, b20,
           We1, be1, W11, b11, W21, b21,
           We2, be2, W12, b12, W22, b22,
           Wout, bout):
    pad = EPAD - E
    src = jnp.concatenate([edge_index[0], jnp.zeros((pad,), jnp.int32)])
    dst = jnp.concatenate([edge_index[1], jnp.full((pad,), N, jnp.int32)])
    sd = jnp.stack([src.reshape(NW, NCHUNK, CH), dst.reshape(NW, NCHUNK, CH)],
                   axis=2)
    ea = jnp.concatenate([edge_attr, jnp.zeros((pad, ED), jnp.float32)])

    layers = [(We0, be0, W10, b10, W20, b20),
              (We1, be1, W11, b11, W21, b21),
              (We2, be2, W12, b12, W22, b22)]
    h = x
    for (we, be, w1, b1, w2, b2) in layers:
        e = _edge_mlp(ea, we, be.reshape(1, D))
        parts = _sc_message(h, e, sd)
        h = _node_mlp(h, parts, w1, b1.reshape(1, D), w2, b2.reshape(1, D))
    out = _final(h, Wout.reshape(1, D), bout.reshape(1, 1))
    return jnp.squeeze(out, axis=-1)
